# Initial kernel scaffold; baseline (speedup 1.0000x reference)
import jax
import jax.numpy as jnp
from jax.experimental import pallas as pl


def kernel(eps_scores, ana_scores, solution_mask):
    raise NotImplementedError("write your pallas kernel here")


# speedup vs baseline: 189.7089x; 189.7089x over previous
import functools

import jax
import jax.numpy as jnp
from jax import lax
from jax.experimental import pallas as pl
from jax.experimental.pallas import tpu as pltpu
from jax.experimental.pallas import tpu_sc as plsc

N = 2048
NUM_WORKERS = 32
GR = 4
GROUPS = 16
EPS_BUF = 2064
ANA_BUF = 8256
ANA_PAD_LEN = 2 * 1024 * 1024
NEG = -1e30

FALSE_NEW = 1.2
FALSE_LINK = 0.5
WRONG_LINK = 1.0


def _sc_body(eps_hbm, ana_hbm, mask_hbm, out_hbm,
             eps_t, m0, m1, a0, a1, row4,
             sm0, sm1, sa0, sa1):
    c = lax.axis_index("c")
    s = lax.axis_index("s")
    wid = s * 2 + c

    pltpu.sync_copy(eps_hbm, eps_t.at[pl.ds(0, N)])
    iota = lax.iota(jnp.int32, 16)
    mbufs, msems = (m0, m1), (sm0, sm1)
    abufs, asems = (a0, a1), (sa0, sa1)

    def issue(g, p):
        i0 = GR * wid + 128 * g
        cols = 128 * (g + 1)
        alen = 512 * (g + 1)
        start = (i0 * (i0 - 1)) // 2
        start8 = (start // 8) * 8
        cm = pltpu.async_copy(
            mask_hbm.at[pl.ds(i0, GR), pl.ds(0, cols)],
            mbufs[p].at[:, pl.ds(0, cols)], msems[p])
        ca = pltpu.async_copy(
            ana_hbm.at[pl.ds(start8, alen)],
            abufs[p].at[pl.ds(0, alen)], asems[p])
        return cm, ca, i0, start8

    pend = issue(0, 0)
    for g in range(GROUPS):
        p = g & 1
        cm, ca, i0, start8 = pend
        cm.wait()
        ca.wait()
        if g + 1 < GROUPS:
            pend = issue(g + 1, 1 - p)
        mb, ab = mbufs[p], abufs[p]
        for k in range(GR):
            i = i0 + k
            off = (i * (i - 1)) // 2 - start8
            trip = (i + 31) // 32

            def col_body(j, carry, i=i, off=off, mb=mb, ab=ab, k=k):
                bsum, mmax = carry
                for h in range(2):
                    base = j * 32 + h * 16
                    col = base + iota
                    mvec = mb[k, pl.ds(base, 16)]
                    avec = ab[pl.ds(off + base, 16)]
                    valid = col < i
                    mv = jnp.where(valid, mvec, 0.0)
                    av = jnp.where(valid, avec, 0.0)
                    bsum = bsum + mv * av
                    keep = valid & (mvec < 0.5)
                    mmax = jnp.maximum(mmax, jnp.where(keep, avec, NEG))
                return bsum, mmax

            init = (jnp.zeros((16,), jnp.float32),
                    jnp.full((16,), NEG, jnp.float32))
            bsum, mmax = lax.fori_loop(0, trip, col_body, init)

            dstart = jnp.maximum(i - 15, 0)
            lane_sel = i - dstart
            dvec = mb[k, pl.ds(dstart, 16)]
            row4[k, pl.ds(0, 16)] = bsum
            row4[k, pl.ds(16, 16)] = mmax
            row4[k, pl.ds(32, 16)] = jnp.where(iota == lane_sel, dvec, 0.0)
            row4[k, pl.ds(48, 16)] = eps_t[pl.ds(i, 16)]
        pltpu.sync_copy(row4, out_hbm.at[pl.ds(i0, GR)])


def _tc_body(buf_ref, out_ref):
    bsum = buf_ref[:, 0:16]
    mmax = buf_ref[:, 16:32]
    dm = jnp.sum(buf_ref[:, 32:48], axis=1, keepdims=True)
    ei = buf_ref[:, 48:49]
    btri = jnp.sum(bsum, axis=1, keepdims=True)
    m = jnp.max(mmax, axis=1, keepdims=True)
    b = btri + dm * ei
    non_ana = dm > 0.5
    cc = jnp.where(non_ana, FALSE_LINK, WRONG_LINK)
    dd = jnp.where(non_ana, NEG, FALSE_NEW * (1.0 + ei - b))
    tt = cc * (1.0 + m - b)
    rl = jnp.maximum(0.0, jnp.maximum(dd, tt))
    out_ref[...] = jnp.broadcast_to(jnp.sum(rl), (1, 1))


@jax.jit
def _run(eps_scores, ana_pad, solution_mask):
    mesh = plsc.VectorSubcoreMesh(core_axis_name="c", subcore_axis_name="s")
    call = functools.partial(
        pl.kernel,
        mesh=mesh,
        compiler_params=pltpu.CompilerParams(use_tc_tiling_on_sc=False),
        out_type=jax.ShapeDtypeStruct((N, 64), jnp.float32),
        scratch_types=[
            pltpu.VMEM((EPS_BUF,), jnp.float32),
            pltpu.VMEM((GR, N), jnp.float32),
            pltpu.VMEM((GR, N), jnp.float32),
            pltpu.VMEM((ANA_BUF,), jnp.float32),
            pltpu.VMEM((ANA_BUF,), jnp.float32),
            pltpu.VMEM((GR, 64), jnp.float32),
            pltpu.SemaphoreType.DMA,
            pltpu.SemaphoreType.DMA,
            pltpu.SemaphoreType.DMA,
            pltpu.SemaphoreType.DMA,
        ],
    )(_sc_body)
    partials = call(eps_scores, ana_pad, solution_mask)
    loss = pl.pallas_call(
        _tc_body,
        out_shape=jax.ShapeDtypeStruct((1, 1), jnp.float32),
    )(partials)
    return loss[0, 0]


def kernel(eps_scores, ana_scores, solution_mask):
    ana_pad = jnp.zeros((ANA_PAD_LEN,), jnp.float32).at[: ana_scores.shape[0]].set(ana_scores)
    return _run(eps_scores, ana_pad, solution_mask)


# speedup vs baseline: 198.4955x; 1.0463x over previous
import functools

import jax
import jax.numpy as jnp
from jax import lax
from jax.experimental import pallas as pl
from jax.experimental.pallas import tpu as pltpu
from jax.experimental.pallas import tpu_sc as plsc

N = 2048
NUM_WORKERS = 32
GR = 8
GROUPS = 8
EPS_BUF = 2064
MASK_W = 2080
ANA_BUF = 16448
ANA_LEN = N * (N - 1) // 2
NEG = -1e30
BIG = 1e33

FALSE_NEW = 1.2
FALSE_LINK = 0.5
WRONG_LINK = 1.0


def _sc_body(eps_hbm, ana_hbm, mask_hbm, out_hbm,
             eps_t, m0, m1, a0, a1, row8,
             sm0, sm1, sa0, sa1):
    c = lax.axis_index("c")
    s = lax.axis_index("s")
    wid = s * 2 + c

    pltpu.sync_copy(eps_hbm, eps_t.at[pl.ds(0, N)])
    iota = lax.iota(jnp.int32, 16)
    mbufs, msems = (m0, m1), (sm0, sm1)
    abufs, asems = (a0, a1), (sa0, sa1)

    def issue(g, p):
        w = wid if g % 2 == 0 else 31 - wid
        i0 = GR * w + 256 * g
        cols = 256 * (g + 1)
        alen = 2048 * (g + 1)
        start = (i0 * (i0 - 1)) // 2
        start8 = jnp.minimum((start // 8) * 8, ANA_LEN - alen)
        cm = pltpu.async_copy(
            mask_hbm.at[pl.ds(i0, GR), pl.ds(0, cols)],
            mbufs[p].at[:, pl.ds(0, cols)], msems[p])
        ca = pltpu.async_copy(
            ana_hbm.at[pl.ds(start8, alen)],
            abufs[p].at[pl.ds(0, alen)], asems[p])
        return cm, ca, i0, start8

    pend = issue(0, 0)
    for g in range(GROUPS):
        p = g & 1
        cm, ca, i0, start8 = pend
        cm.wait()
        ca.wait()
        if g + 1 < GROUPS:
            pend = issue(g + 1, 1 - p)
        mb, ab = mbufs[p], abufs[p]
        for k in reversed(range(GR)):
            i = i0 + k
            off = (i * (i - 1)) // 2 - start8

            dstart = jnp.maximum(i - 15, 0)
            lane_sel = i - dstart
            dvec = mb[k, pl.ds(dstart, 16)]

            zeros16 = jnp.zeros((16,), jnp.float32)
            negs16 = jnp.full((16,), NEG, jnp.float32)
            mb[k, pl.ds(i, 16)] = zeros16
            mb[k, pl.ds(i + 16, 16)] = zeros16
            ab[pl.ds(off + i, 16)] = negs16
            ab[pl.ds(off + i + 16, 16)] = negs16

            trip = (i + 31) // 32

            def col_body(j, carry, i=i, off=off, mb=mb, ab=ab, k=k):
                bsum, mmax = carry
                for h in range(2):
                    base = j * 32 + h * 16
                    mvec = mb[k, pl.ds(base, 16)]
                    avec = ab[pl.ds(off + base, 16)]
                    bsum = bsum + mvec * avec
                    mmax = jnp.maximum(mmax, avec - BIG * mvec)
                return bsum, mmax

            init = (jnp.zeros((16,), jnp.float32),
                    jnp.full((16,), NEG, jnp.float32))
            bsum, mmax = lax.fori_loop(0, trip, col_body, init)

            row8[k, pl.ds(0, 16)] = bsum
            row8[k, pl.ds(16, 16)] = mmax
            row8[k, pl.ds(32, 16)] = jnp.where(iota == lane_sel, dvec, 0.0)
            row8[k, pl.ds(48, 16)] = eps_t[pl.ds(i, 16)]
        pltpu.sync_copy(row8, out_hbm.at[pl.ds(i0, GR)])


def _tc_body(buf_ref, out_ref):
    bsum = buf_ref[:, 0:16]
    mmax = buf_ref[:, 16:32]
    dm = jnp.sum(buf_ref[:, 32:48], axis=1, keepdims=True)
    ei = buf_ref[:, 48:49]
    btri = jnp.sum(bsum, axis=1, keepdims=True)
    m = jnp.max(mmax, axis=1, keepdims=True)
    b = btri + dm * ei
    non_ana = dm > 0.5
    cc = jnp.where(non_ana, FALSE_LINK, WRONG_LINK)
    dd = jnp.where(non_ana, NEG, FALSE_NEW * (1.0 + ei - b))
    tt = cc * (1.0 + m - b)
    rl = jnp.maximum(0.0, jnp.maximum(dd, tt))
    out_ref[...] = jnp.broadcast_to(jnp.sum(rl), (1, 1))


@jax.jit
def _run(eps_scores, ana_scores, solution_mask):
    mesh = plsc.VectorSubcoreMesh(core_axis_name="c", subcore_axis_name="s")
    call = functools.partial(
        pl.kernel,
        mesh=mesh,
        compiler_params=pltpu.CompilerParams(use_tc_tiling_on_sc=False),
        out_type=jax.ShapeDtypeStruct((N, 64), jnp.float32),
        scratch_types=[
            pltpu.VMEM((EPS_BUF,), jnp.float32),
            pltpu.VMEM((GR, MASK_W), jnp.float32),
            pltpu.VMEM((GR, MASK_W), jnp.float32),
            pltpu.VMEM((ANA_BUF,), jnp.float32),
            pltpu.VMEM((ANA_BUF,), jnp.float32),
            pltpu.VMEM((GR, 64), jnp.float32),
            pltpu.SemaphoreType.DMA,
            pltpu.SemaphoreType.DMA,
            pltpu.SemaphoreType.DMA,
            pltpu.SemaphoreType.DMA,
        ],
    )(_sc_body)
    partials = call(eps_scores, ana_scores, solution_mask)
    loss = pl.pallas_call(
        _tc_body,
        out_shape=jax.ShapeDtypeStruct((1, 1), jnp.float32),
    )(partials)
    return loss[0, 0]


def kernel(eps_scores, ana_scores, solution_mask):
    return _run(eps_scores, ana_scores, solution_mask)


# speedup vs baseline: 276.4278x; 1.3926x over previous
import functools

import jax
import jax.numpy as jnp
from jax import lax
from jax.experimental import pallas as pl
from jax.experimental.pallas import tpu as pltpu
from jax.experimental.pallas import tpu_sc as plsc

N = 2048
NUM_WORKERS = 32
GR = 8
GROUPS = 8
CB = 512
ANA_BUF = 16512
ANA_LEN = N * (N - 1) // 2
NEG = -1e30

FALSE_NEW = 1.2
FALSE_LINK = 0.5
WRONG_LINK = 1.0


def _tc_ante_body(mask_ref, ante_ref, gidx_ref):
    g = pl.program_id(0)
    colv = (lax.broadcasted_iota(jnp.int32, (N, CB), 1) + g * CB).astype(jnp.float32)
    part = jnp.sum(mask_ref[...] * colv, axis=1, keepdims=True)

    @pl.when(g == 0)
    def _():
        ante_ref[...] = part

    @pl.when(g > 0)
    def _():
        ante_ref[...] = ante_ref[...] + part

    @pl.when(g == pl.num_programs(0) - 1)
    def _():
        rowv = lax.broadcasted_iota(jnp.int32, (N, 1), 0).astype(jnp.float32)
        tri = rowv * (rowv - 1.0) * 0.5
        gidx_ref[...] = (tri + ante_ref[...]).astype(jnp.int32)


def _sc_body(ana_hbm, gidx_hbm, out_hbm, idx_t, a0, a1, row8, sa0, sa1):
    c = lax.axis_index("c")
    s = lax.axis_index("s")
    wid = s * 2 + c

    iota = lax.iota(jnp.int32, 16)
    negs16 = jnp.full((16,), NEG, jnp.float32)
    abufs, asems = (a0, a1), (sa0, sa1)

    def group_info(g):
        w = wid if g % 2 == 0 else 31 - wid
        i0 = GR * w + 256 * g
        alen = 2048 * (g + 1)
        start = (i0 * (i0 - 1)) // 2
        start8 = jnp.minimum((start // 8) * 8, ANA_LEN - alen)
        return i0, alen, start8

    for g in range(GROUPS):
        i0, _, _ = group_info(g)
        pltpu.sync_copy(gidx_hbm.at[pl.ds(i0, GR)], idx_t.at[pl.ds(GR * g, GR)])

    def issue(g, p):
        i0, alen, start8 = group_info(g)
        ca = pltpu.async_copy(
            ana_hbm.at[pl.ds(start8, alen)],
            abufs[p].at[pl.ds(0, alen)], asems[p])
        return ca, i0, start8

    pend = issue(0, 0)
    for g in range(GROUPS):
        p = g & 1
        ca, i0, start8 = pend
        ca.wait()
        if g + 1 < GROUPS:
            pend = issue(g + 1, 1 - p)
        ab = abufs[p]
        for k in reversed(range(GR)):
            i = i0 + k
            off = (i * (i - 1)) // 2 - start8
            r = GR * g + k
            q = idx_t[pl.ds((r // 16) * 16, 16)][r % 16] - start8

            bvec = ab[pl.ds(q, 16)]
            ab[pl.ds(q, 16)] = jnp.where(iota == 0, NEG, bvec)
            for h in range(4):
                ab[pl.ds(off + i + 16 * h, 16)] = negs16

            trip = (i + 63) // 64

            def col_body(j, mmax, off=off, ab=ab):
                for h in range(4):
                    mmax = jnp.maximum(mmax, ab[pl.ds(off + j * 64 + h * 16, 16)])
                return mmax

            mmax = lax.fori_loop(0, trip, col_body, negs16)

            row8[k, pl.ds(0, 16)] = mmax
            row8[k, pl.ds(16, 16)] = bvec
        pltpu.sync_copy(row8, out_hbm.at[pl.ds(i0, GR)])


def _tc_comb_body(buf_ref, ante_ref, eps_ref, out_ref):
    m = jnp.max(buf_ref[:, 0:16], axis=1, keepdims=True)
    bg = buf_ref[:, 16:17]
    ante = ante_ref[...]
    ei = eps_ref[...]
    rowv = lax.broadcasted_iota(jnp.int32, (N, 1), 0).astype(jnp.float32)
    non_ana = ante == rowv
    b = jnp.where(non_ana, ei, bg)
    cc = jnp.where(non_ana, FALSE_LINK, WRONG_LINK)
    dd = jnp.where(non_ana, NEG, FALSE_NEW * (1.0 + ei - b))
    tt = cc * (1.0 + m - b)
    rl = jnp.maximum(0.0, jnp.maximum(dd, tt))
    out_ref[...] = jnp.broadcast_to(jnp.sum(rl), (1, 1))


@jax.jit
def _run(eps_scores, ana_scores, solution_mask):
    antef, gidx = pl.pallas_call(
        _tc_ante_body,
        grid=(N // CB,),
        in_specs=[pl.BlockSpec((N, CB), lambda g: (0, g))],
        out_specs=[pl.BlockSpec((N, 1), lambda g: (0, 0)),
                   pl.BlockSpec((N, 1), lambda g: (0, 0))],
        out_shape=[jax.ShapeDtypeStruct((N, 1), jnp.float32),
                   jax.ShapeDtypeStruct((N, 1), jnp.int32)],
    )(solution_mask)

    mesh = plsc.VectorSubcoreMesh(core_axis_name="c", subcore_axis_name="s")
    call = functools.partial(
        pl.kernel,
        mesh=mesh,
        compiler_params=pltpu.CompilerParams(use_tc_tiling_on_sc=False),
        out_type=jax.ShapeDtypeStruct((N, 32), jnp.float32),
        scratch_types=[
            pltpu.VMEM((64,), jnp.int32),
            pltpu.VMEM((ANA_BUF,), jnp.float32),
            pltpu.VMEM((ANA_BUF,), jnp.float32),
            pltpu.VMEM((GR, 32), jnp.float32),
            pltpu.SemaphoreType.DMA,
            pltpu.SemaphoreType.DMA,
        ],
    )(_sc_body)
    partials = call(ana_scores, gidx.reshape(N))

    loss = pl.pallas_call(
        _tc_comb_body,
        out_shape=jax.ShapeDtypeStruct((1, 1), jnp.float32),
    )(partials, antef, eps_scores.reshape(N, 1))
    return loss[0, 0]


def kernel(eps_scores, ana_scores, solution_mask):
    return _run(eps_scores, ana_scores, solution_mask)
